# baseline (device time: 27276 ns/iter reference)
import jax
import jax.numpy as jnp
from jax import lax
from jax.experimental import pallas as pl
from jax.experimental.pallas import tpu as pltpu

N_DEV = 4
B, SQ, D = 2, 256, 768
H_LOC, DH = 8, 64
SKV = 512
HD = H_LOC * DH
ROWS = B * SQ
QR = ROWS // N_DEV

F32 = jnp.float32
BF16 = jnp.bfloat16

COMM = False


def kernel(x, Wq, Wo, K_ext, V_ext):
    my_i = lax.axis_index("i")
    Kl = lax.dynamic_slice_in_dim(
        K_ext.reshape(B, SKV, 32 * DH), my_i * HD, HD, axis=2).astype(BF16)
    Vl = lax.dynamic_slice_in_dim(
        V_ext.reshape(B, SKV, 32 * DH), my_i * HD, HD, axis=2).astype(BF16)

    def body(x_ref, wq_ref, wo_ref, k_ref, v_ref, out_ref,
             q_ref, attn_ref, pacc_ref, rs_ref, redq_ref, ag_ref,
             rs_send_sems, rs_recv_sems, ag_send_sems, ag_recv_sems):
        my_i = lax.axis_index("i")

        if COMM:
            barrier = pltpu.get_barrier_semaphore()
            for delta in range(1, N_DEV):
                peer = lax.rem(my_i + delta, N_DEV)
                pl.semaphore_signal(barrier, inc=1, device_id=(peer,),
                                    device_id_type=pl.DeviceIdType.MESH)
            pl.semaphore_wait(barrier, N_DEV - 1)

        wq16 = (wq_ref[...] * 0.125).astype(BF16)
        for b in range(B):
            q_ref[b * SQ:(b + 1) * SQ, :] = jnp.dot(
                x_ref[b].astype(BF16), wq16,
                preferred_element_type=F32).astype(BF16)
        wo16 = wo_ref[...].astype(BF16)

        def half_attention(bsel):
            row = pl.ds(bsel * SQ, SQ)
            for h in range(H_LOC):
                hcol = slice(h * DH, (h + 1) * DH)
                q = q_ref[row, hcol]
                k = k_ref[pl.ds(bsel, 1), :, hcol].reshape(SKV, DH)
                v = v_ref[pl.ds(bsel, 1), :, hcol].reshape(SKV, DH)
                s = lax.dot_general(
                    q, k, (((1,), (1,)), ((), ())),
                    preferred_element_type=F32)
                p = jnp.exp(s)
                l = jnp.sum(p, axis=1, keepdims=True)
                o = jnp.dot(p.astype(BF16), v,
                            preferred_element_type=F32) / l
                attn_ref[row, hcol] = o.astype(BF16)

        def wo_quarter(c):
            return jnp.dot(attn_ref[pl.ds(c * QR, QR), :], wo16,
                           preferred_element_type=F32)

        def out_store(c, val):
            bq = lax.div(c, 2)
            off = lax.rem(c, 2) * QR
            out_ref[pl.ds(bq, 1), pl.ds(off, QR), :] = val.reshape(1, QR, D)

        sends = []

        def send_quarter(delta):
            peer = lax.rem(my_i + delta, N_DEV)
            pacc_ref[pl.ds(peer * QR, QR), :] = \
                wo_quarter(peer).astype(BF16)
            if not COMM:
                return
            rdma = pltpu.make_async_remote_copy(
                src_ref=pacc_ref.at[pl.ds(peer * QR, QR)],
                dst_ref=rs_ref.at[N_DEV - 1 - delta],
                send_sem=rs_send_sems.at[delta - 1],
                recv_sem=rs_recv_sems.at[N_DEV - 1 - delta],
                device_id=(peer,),
                device_id_type=pl.DeviceIdType.MESH,
            )
            rdma.start()
            sends.append(rdma)

        first_half = lax.div(lax.rem(my_i + 1, N_DEV), 2)
        half_attention(first_half)
        send_quarter(1)
        half_attention(1 - first_half)
        send_quarter(2)
        send_quarter(3)

        mine = wo_quarter(my_i)

        if not COMM:
            out_store(my_i, mine)
            for delta in range(1, N_DEV):
                peer = lax.rem(my_i + delta, N_DEV)
                out_store(peer, pacc_ref[pl.ds(peer * QR, QR), :].astype(F32))
            return

        red = mine
        for slot in range(N_DEV - 2, -1, -1):
            recv = pltpu.make_async_remote_copy(
                src_ref=rs_ref.at[slot], dst_ref=rs_ref.at[slot],
                send_sem=rs_send_sems.at[0],
                recv_sem=rs_recv_sems.at[slot],
                device_id=(my_i,), device_id_type=pl.DeviceIdType.MESH,
            )
            recv.wait_recv()
            red = red + rs_ref[slot, :, :].astype(F32)
        out_store(my_i, red)
        redq_ref[...] = red.astype(BF16)

        for delta in (2, 1, 3):
            peer = lax.rem(my_i + delta, N_DEV)
            rdma = pltpu.make_async_remote_copy(
                src_ref=redq_ref,
                dst_ref=ag_ref.at[N_DEV - 1 - delta],
                send_sem=ag_send_sems.at[delta - 1],
                recv_sem=ag_recv_sems.at[N_DEV - 1 - delta],
                device_id=(peer,),
                device_id_type=pl.DeviceIdType.MESH,
            )
            rdma.start()
            sends.append(rdma)

        for slot in range(N_DEV - 1):
            recv = pltpu.make_async_remote_copy(
                src_ref=ag_ref.at[slot], dst_ref=ag_ref.at[slot],
                send_sem=ag_send_sems.at[0],
                recv_sem=ag_recv_sems.at[slot],
                device_id=(my_i,), device_id_type=pl.DeviceIdType.MESH,
            )
            recv.wait_recv()
            sender = lax.rem(my_i + slot + 1, N_DEV)
            out_store(sender, ag_ref[slot, :, :].astype(F32))

        for rdma in sends:
            rdma.wait_send()

    return pl.pallas_call(
        body,
        out_shape=jax.ShapeDtypeStruct((B, SQ, D), F32),
        in_specs=[pl.BlockSpec(memory_space=pltpu.VMEM)] * 5,
        out_specs=pl.BlockSpec(memory_space=pltpu.VMEM),
        scratch_shapes=[
            pltpu.VMEM((ROWS, HD), BF16),
            pltpu.VMEM((ROWS, HD), BF16),
            pltpu.VMEM((ROWS, D), BF16),
            pltpu.VMEM((N_DEV - 1, QR, D), BF16),
            pltpu.VMEM((QR, D), BF16),
            pltpu.VMEM((N_DEV - 1, QR, D), BF16),
            pltpu.SemaphoreType.DMA((N_DEV - 1,)),
            pltpu.SemaphoreType.DMA((N_DEV - 1,)),
            pltpu.SemaphoreType.DMA((N_DEV - 1,)),
            pltpu.SemaphoreType.DMA((N_DEV - 1,)),
        ],
        compiler_params=(pltpu.CompilerParams(collective_id=0)
                         if COMM else None),
    )(x, Wq, Wo, Kl, Vl)


# device time: 25757 ns/iter; 1.0590x vs baseline; 1.0590x over previous
import jax
import jax.numpy as jnp
from jax import lax
from jax.experimental import pallas as pl
from jax.experimental.pallas import tpu as pltpu

N_DEV = 4
B, SQ, D = 2, 256, 768
H_LOC, DH = 8, 64
SKV = 512
HD = H_LOC * DH
ROWS = B * SQ
QR = ROWS // N_DEV

F32 = jnp.float32
BF16 = jnp.bfloat16

COMM = False


def kernel(x, Wq, Wo, K_ext, V_ext):
    my_i = lax.axis_index("i")
    Kl = lax.dynamic_slice_in_dim(
        K_ext.reshape(B, SKV, 32 * DH), my_i * HD, HD, axis=2).astype(BF16)
    Vl = lax.dynamic_slice_in_dim(
        V_ext.reshape(B, SKV, 32 * DH), my_i * HD, HD, axis=2).astype(BF16)

    def body(x_ref, wq_ref, wo_ref, k_ref, v_ref, out_ref,
             q_ref, aq_ref, pacc_ref, rs_ref, redq_ref, ag_ref,
             rs_send_sems, rs_recv_sems, ag_send_sems, ag_recv_sems):
        my_i = lax.axis_index("i")

        if COMM:
            barrier = pltpu.get_barrier_semaphore()
            for delta in range(1, N_DEV):
                peer = lax.rem(my_i + delta, N_DEV)
                pl.semaphore_signal(barrier, inc=1, device_id=(peer,),
                                    device_id_type=pl.DeviceIdType.MESH)
            pl.semaphore_wait(barrier, N_DEV - 1)

        wq16 = (wq_ref[...] * 0.125).astype(BF16)
        for b in range(B):
            q_ref[b * SQ:(b + 1) * SQ, :] = jnp.dot(
                x_ref[b].astype(BF16), wq16,
                preferred_element_type=F32).astype(BF16)
        wo16 = wo_ref[...].astype(BF16)

        def quarter_partial(c):
            b = lax.div(c, 2)
            for h in range(H_LOC):
                hcol = slice(h * DH, (h + 1) * DH)
                q = q_ref[pl.ds(c * QR, QR), hcol]
                k = k_ref[pl.ds(b, 1), :, hcol].reshape(SKV, DH)
                v = v_ref[pl.ds(b, 1), :, hcol].reshape(SKV, DH)
                s = lax.dot_general(
                    q, k, (((1,), (1,)), ((), ())),
                    preferred_element_type=F32)
                p = jnp.exp(s)
                l = jnp.sum(p, axis=1, keepdims=True)
                o = jnp.dot(p.astype(BF16), v,
                            preferred_element_type=F32) / l
                aq_ref[:, hcol] = o.astype(BF16)
            return jnp.dot(aq_ref[...], wo16,
                           preferred_element_type=F32)

        def out_store(c, val):
            bq = lax.div(c, 2)
            off = lax.rem(c, 2) * QR
            out_ref[pl.ds(bq, 1), pl.ds(off, QR), :] = val.reshape(1, QR, D)

        sends = []
        for delta in range(1, N_DEV):
            peer = lax.rem(my_i + delta, N_DEV)
            pacc_ref[pl.ds(peer * QR, QR), :] = \
                quarter_partial(peer).astype(BF16)
            if not COMM:
                continue
            rdma = pltpu.make_async_remote_copy(
                src_ref=pacc_ref.at[pl.ds(peer * QR, QR)],
                dst_ref=rs_ref.at[N_DEV - 1 - delta],
                send_sem=rs_send_sems.at[delta - 1],
                recv_sem=rs_recv_sems.at[N_DEV - 1 - delta],
                device_id=(peer,),
                device_id_type=pl.DeviceIdType.MESH,
            )
            rdma.start()
            sends.append(rdma)

        mine = quarter_partial(my_i)

        if not COMM:
            out_store(my_i, mine)
            for delta in range(1, N_DEV):
                peer = lax.rem(my_i + delta, N_DEV)
                out_store(peer, pacc_ref[pl.ds(peer * QR, QR), :].astype(F32))
            return

        for slot in range(N_DEV - 1):
            recv = pltpu.make_async_remote_copy(
                src_ref=rs_ref.at[slot], dst_ref=rs_ref.at[slot],
                send_sem=rs_send_sems.at[0],
                recv_sem=rs_recv_sems.at[slot],
                device_id=(my_i,), device_id_type=pl.DeviceIdType.MESH,
            )
            recv.wait_recv()
        red = (mine + rs_ref[0, :, :].astype(F32)
               + rs_ref[1, :, :].astype(F32)
               + rs_ref[2, :, :].astype(F32))
        out_store(my_i, red)
        redq_ref[...] = red.astype(BF16)

        for delta in range(1, N_DEV):
            peer = lax.rem(my_i + delta, N_DEV)
            rdma = pltpu.make_async_remote_copy(
                src_ref=redq_ref,
                dst_ref=ag_ref.at[N_DEV - 1 - delta],
                send_sem=ag_send_sems.at[delta - 1],
                recv_sem=ag_recv_sems.at[N_DEV - 1 - delta],
                device_id=(peer,),
                device_id_type=pl.DeviceIdType.MESH,
            )
            rdma.start()
            sends.append(rdma)

        for slot in range(N_DEV - 1):
            recv = pltpu.make_async_remote_copy(
                src_ref=ag_ref.at[slot], dst_ref=ag_ref.at[slot],
                send_sem=ag_send_sems.at[0],
                recv_sem=ag_recv_sems.at[slot],
                device_id=(my_i,), device_id_type=pl.DeviceIdType.MESH,
            )
            recv.wait_recv()
            sender = lax.rem(my_i + slot + 1, N_DEV)
            out_store(sender, ag_ref[slot, :, :].astype(F32))

        for rdma in sends:
            rdma.wait_send()

    return pl.pallas_call(
        body,
        out_shape=jax.ShapeDtypeStruct((B, SQ, D), F32),
        in_specs=[pl.BlockSpec(memory_space=pltpu.VMEM)] * 5,
        out_specs=pl.BlockSpec(memory_space=pltpu.VMEM),
        scratch_shapes=[
            pltpu.VMEM((ROWS, HD), BF16),
            pltpu.VMEM((QR, HD), BF16),
            pltpu.VMEM((ROWS, D), BF16),
            pltpu.VMEM((N_DEV - 1, QR, D), BF16),
            pltpu.VMEM((QR, D), BF16),
            pltpu.VMEM((N_DEV - 1, QR, D), BF16),
            pltpu.SemaphoreType.DMA((N_DEV - 1,)),
            pltpu.SemaphoreType.DMA((N_DEV - 1,)),
            pltpu.SemaphoreType.DMA((N_DEV - 1,)),
            pltpu.SemaphoreType.DMA((N_DEV - 1,)),
        ],
        compiler_params=(pltpu.CompilerParams(collective_id=0)
                         if COMM else None),
    )(x, Wq, Wo, Kl, Vl)


# device time: 25507 ns/iter; 1.0694x vs baseline; 1.0098x over previous
import jax
import jax.numpy as jnp
from jax import lax
from jax.experimental import pallas as pl
from jax.experimental.pallas import tpu as pltpu

N_DEV = 4
B, SQ, D = 2, 256, 768
H_LOC, DH = 8, 64
SKV = 512
HD = H_LOC * DH
ROWS = B * SQ
QR = ROWS // N_DEV

F32 = jnp.float32
BF16 = jnp.bfloat16

COMM = False


def kernel(x, Wq, Wo, K_ext, V_ext):
    my_i = lax.axis_index("i")
    Kl = lax.dynamic_slice_in_dim(
        K_ext.reshape(B, SKV, 32 * DH), my_i * HD, HD, axis=2).astype(BF16)
    Vl = lax.dynamic_slice_in_dim(
        V_ext.reshape(B, SKV, 32 * DH), my_i * HD, HD, axis=2).astype(BF16)

    def body(x_ref, wq_ref, wo_ref, k_ref, v_ref, out_ref,
             q_ref, aq_ref, pacc_ref, rs_ref, redq_ref, ag_ref,
             rs_send_sems, rs_recv_sems, ag_send_sems, ag_recv_sems):
        my_i = lax.axis_index("i")

        if COMM:
            barrier = pltpu.get_barrier_semaphore()
            for delta in range(1, N_DEV):
                peer = lax.rem(my_i + delta, N_DEV)
                pl.semaphore_signal(barrier, inc=1, device_id=(peer,),
                                    device_id_type=pl.DeviceIdType.MESH)

        wq16 = (wq_ref[...] * 0.125).astype(BF16)
        for b in range(B):
            q_ref[b * SQ:(b + 1) * SQ, :] = jnp.dot(
                x_ref[b].astype(BF16), wq16,
                preferred_element_type=F32).astype(BF16)
        wo16 = wo_ref[...].astype(BF16)

        if COMM:
            pl.semaphore_wait(barrier, N_DEV - 1)

        def quarter_partial(c):
            b = lax.div(c, 2)
            for h in range(H_LOC):
                hcol = slice(h * DH, (h + 1) * DH)
                q = q_ref[pl.ds(c * QR, QR), hcol]
                k = k_ref[pl.ds(b, 1), :, hcol].reshape(SKV, DH)
                v = v_ref[pl.ds(b, 1), :, hcol].reshape(SKV, DH)
                s = lax.dot_general(
                    q, k, (((1,), (1,)), ((), ())),
                    preferred_element_type=F32)
                p = jnp.exp(s)
                l = jnp.sum(p, axis=1, keepdims=True)
                o = jnp.dot(p.astype(BF16), v,
                            preferred_element_type=F32) / l
                aq_ref[:, hcol] = o.astype(BF16)
            return jnp.dot(aq_ref[...], wo16,
                           preferred_element_type=F32)

        def out_store(c, val):
            bq = lax.div(c, 2)
            off = lax.rem(c, 2) * QR
            out_ref[pl.ds(bq, 1), pl.ds(off, QR), :] = val.reshape(1, QR, D)

        sends = []
        for delta in range(1, N_DEV):
            peer = lax.rem(my_i + delta, N_DEV)
            pacc_ref[pl.ds(peer * QR, QR), :] = \
                quarter_partial(peer).astype(BF16)
            if not COMM:
                continue
            rdma = pltpu.make_async_remote_copy(
                src_ref=pacc_ref.at[pl.ds(peer * QR, QR)],
                dst_ref=rs_ref.at[N_DEV - 1 - delta],
                send_sem=rs_send_sems.at[delta - 1],
                recv_sem=rs_recv_sems.at[N_DEV - 1 - delta],
                device_id=(peer,),
                device_id_type=pl.DeviceIdType.MESH,
            )
            rdma.start()
            sends.append(rdma)

        mine = quarter_partial(my_i)

        if not COMM:
            out_store(my_i, mine)
            for delta in range(1, N_DEV):
                peer = lax.rem(my_i + delta, N_DEV)
                out_store(peer, pacc_ref[pl.ds(peer * QR, QR), :].astype(F32))
            return

        for slot in range(N_DEV - 1):
            recv = pltpu.make_async_remote_copy(
                src_ref=rs_ref.at[slot], dst_ref=rs_ref.at[slot],
                send_sem=rs_send_sems.at[0],
                recv_sem=rs_recv_sems.at[slot],
                device_id=(my_i,), device_id_type=pl.DeviceIdType.MESH,
            )
            recv.wait_recv()
        red = (mine + rs_ref[0, :, :].astype(F32)
               + rs_ref[1, :, :].astype(F32)
               + rs_ref[2, :, :].astype(F32))
        out_store(my_i, red)
        redq_ref[...] = red.astype(BF16)

        for delta in range(1, N_DEV):
            peer = lax.rem(my_i + delta, N_DEV)
            rdma = pltpu.make_async_remote_copy(
                src_ref=redq_ref,
                dst_ref=ag_ref.at[N_DEV - 1 - delta],
                send_sem=ag_send_sems.at[delta - 1],
                recv_sem=ag_recv_sems.at[N_DEV - 1 - delta],
                device_id=(peer,),
                device_id_type=pl.DeviceIdType.MESH,
            )
            rdma.start()
            sends.append(rdma)

        for slot in range(N_DEV - 1):
            recv = pltpu.make_async_remote_copy(
                src_ref=ag_ref.at[slot], dst_ref=ag_ref.at[slot],
                send_sem=ag_send_sems.at[0],
                recv_sem=ag_recv_sems.at[slot],
                device_id=(my_i,), device_id_type=pl.DeviceIdType.MESH,
            )
            recv.wait_recv()
            sender = lax.rem(my_i + slot + 1, N_DEV)
            out_store(sender, ag_ref[slot, :, :].astype(F32))

        for rdma in sends:
            rdma.wait_send()

    return pl.pallas_call(
        body,
        out_shape=jax.ShapeDtypeStruct((B, SQ, D), F32),
        in_specs=[pl.BlockSpec(memory_space=pltpu.VMEM)] * 5,
        out_specs=pl.BlockSpec(memory_space=pltpu.VMEM),
        scratch_shapes=[
            pltpu.VMEM((ROWS, HD), BF16),
            pltpu.VMEM((QR, HD), BF16),
            pltpu.VMEM((ROWS, D), BF16),
            pltpu.VMEM((N_DEV - 1, QR, D), BF16),
            pltpu.VMEM((QR, D), BF16),
            pltpu.VMEM((N_DEV - 1, QR, D), BF16),
            pltpu.SemaphoreType.DMA((N_DEV - 1,)),
            pltpu.SemaphoreType.DMA((N_DEV - 1,)),
            pltpu.SemaphoreType.DMA((N_DEV - 1,)),
            pltpu.SemaphoreType.DMA((N_DEV - 1,)),
        ],
        compiler_params=(pltpu.CompilerParams(collective_id=0)
                         if COMM else None),
    )(x, Wq, Wo, Kl, Vl)
